# baseline (speedup 1.0000x reference)
import functools

import jax
import jax.numpy as jnp
from jax.experimental import pallas as pl

_BATCH = 16384
_NPOS = 200
_BB = 512


def _body(pos_ref, emb_ref, out_ref):
    pos = pos_ref[...]
    m = jnp.max(pos, axis=1, keepdims=True)
    col = jax.lax.broadcasted_iota(jnp.int32, pos.shape, 1)
    cand = jnp.where(pos == m, col, _NPOS)
    idx = jnp.min(cand, axis=1, keepdims=True)
    emb = emb_ref[...]
    val = jnp.max(jnp.where(col == idx, emb, -jnp.inf), axis=1, keepdims=True)
    out_ref[...] = val


@functools.partial(jax.jit, static_argnames=())
def kernel(position, embedding):
    emb2d = embedding.reshape(1, _NPOS)
    out = pl.pallas_call(
        _body,
        grid=(_BATCH // _BB,),
        in_specs=[
            pl.BlockSpec((_BB, _NPOS), lambda i: (i, 0)),
            pl.BlockSpec((1, _NPOS), lambda i: (0, 0)),
        ],
        out_specs=pl.BlockSpec((_BB, 1), lambda i: (i, 0)),
        out_shape=jax.ShapeDtypeStruct((_BATCH, 1), jnp.float32),
    )(position, emb2d)
    return out.reshape(_BATCH)


# speedup vs baseline: 1.4463x; 1.4463x over previous
import functools

import jax
import jax.numpy as jnp
from jax.experimental import pallas as pl

_BATCH = 16384
_NPOS = 200
_BB = 4096


def _body(pos_ref, emb_ref, out_ref):
    pos = pos_ref[...]
    m = jnp.max(pos, axis=1, keepdims=True)
    col = jax.lax.broadcasted_iota(jnp.int32, pos.shape, 1)
    cand = jnp.where(pos == m, col, _NPOS)
    idx = jnp.min(cand, axis=1, keepdims=True)
    emb = emb_ref[...]
    val = jnp.max(jnp.where(col == idx, emb, -jnp.inf), axis=1, keepdims=True)
    out_ref[...] = val


@functools.partial(jax.jit, static_argnames=())
def kernel(position, embedding):
    emb2d = embedding.reshape(1, _NPOS)
    out = pl.pallas_call(
        _body,
        grid=(_BATCH // _BB,),
        in_specs=[
            pl.BlockSpec((_BB, _NPOS), lambda i: (i, 0)),
            pl.BlockSpec((1, _NPOS), lambda i: (0, 0)),
        ],
        out_specs=pl.BlockSpec((_BB, 1), lambda i: (i, 0)),
        out_shape=jax.ShapeDtypeStruct((_BATCH, 1), jnp.float32),
    )(position, emb2d)
    return out.reshape(_BATCH)
